# baseline (speedup 1.0000x reference)
import functools

import jax
import jax.numpy as jnp
from jax import lax
from jax.experimental import pallas as pl
from jax.experimental.pallas import tpu as pltpu
from jax.experimental.pallas import tpu_sc as plsc

N = 10000
E = 320000
KER = 8
OUT = 64
NC = 2
NS = 16
NW = NC * NS
EPW = E // NW
C = 40
NCH = EPW // C
CP = 80
NCHP = EPW // CP
NP = 10240
ROWS_T = NP // NS
EPWP = 10240
NCHA = EPWP // C

_mesh = plsc.VectorSubcoreMesh(
    core_axis_name="c", subcore_axis_name="s", num_cores=NC, num_subcores=NS)


def _sc_prep_body(spec_hbm, src2p_hbm, dst2p_hbm, zeros1_hbm,
                  pseudo_hbm, deg_hbm,
                  spec_v, src1_v, dst1_v, pbuf_v, ones_v, deg_sh,
                  dsem):
    cid = lax.axis_index("c")
    sid = lax.axis_index("s")
    wid = cid * NS + sid

    r0 = sid * ROWS_T
    pltpu.sync_copy(zeros1_hbm, deg_sh.at[pl.ds(r0, ROWS_T)])

    pltpu.sync_copy(src2p_hbm.at[wid], src1_v)
    pltpu.sync_copy(dst2p_hbm.at[wid], dst1_v)
    pltpu.sync_copy(spec_hbm, spec_v)

    for i in range(CP // 16):
        ones_v[pl.ds(i * 16, 16)] = jnp.ones((16,), jnp.float32)

    plsc.subcore_barrier()

    def group(g, carry):
        r = g // (CP // 16)
        c0 = (g % (CP // 16)) * 16
        sv = src1_v[r, pl.ds(c0, 16)] * 3
        dv = dst1_v[r, pl.ds(c0, 16)] * 3
        for c in range(3):
            cc = jnp.full((16,), c, jnp.int32)
            ss = plsc.load_gather(spec_v, [sv + cc])
            sd = plsc.load_gather(spec_v, [dv + cc])
            pbuf_v[c, pl.ds(g * 16, 16)] = sd - ss
        return carry

    lax.fori_loop(0, EPW // 16, group, 0)
    pltpu.sync_copy(pbuf_v, pseudo_hbm.at[wid, :, pl.ds(0, EPW)])

    def dchunk(j, carry):
        pltpu.async_copy(ones_v, deg_sh.at[dst1_v.at[j]], dsem, add=True)

        @pl.when(j >= 8)
        def _():
            pltpu.make_async_copy(ones_v, deg_sh.at[dst1_v.at[0]],
                                  dsem).wait()

        return carry

    lax.fori_loop(0, NCHP, dchunk, 0)

    def ddrain(j, carry):
        pltpu.make_async_copy(ones_v, deg_sh.at[dst1_v.at[0]], dsem).wait()
        return carry

    lax.fori_loop(0, 8, ddrain, 0)
    plsc.subcore_barrier()

    pltpu.sync_copy(deg_sh.at[pl.ds(r0, ROWS_T)],
                    deg_hbm.at[cid, 0, pl.ds(r0, ROWS_T)])


_sc_prep = functools.partial(
    pl.kernel,
    compiler_params=pltpu.CompilerParams(needs_layout_passes=False,
                                         use_tc_tiling_on_sc=False),
    out_type=[jax.ShapeDtypeStruct((NW, 3, EPWP), jnp.float32),
              jax.ShapeDtypeStruct((NC, 1, NP), jnp.float32)],
    mesh=_mesh,
    scratch_types=[
        pltpu.VMEM((3 * N,), jnp.float32),
        pltpu.VMEM((NCHP, CP), jnp.int32),
        pltpu.VMEM((NCHP, CP), jnp.int32),
        pltpu.VMEM((3, EPW), jnp.float32),
        pltpu.VMEM((CP,), jnp.float32),
        pltpu.VMEM_SHARED((NP,), jnp.float32),
        pltpu.SemaphoreType.DMA,
    ],
)(_sc_prep_body)


def _sc_edge_body(y_hbm, src2_hbm, dst2_hbm, alpha_hbm, lidx_hbm, zerosm_hbm,
                  agg_hbm,
                  src2_v, dst2_v, lidx_v, alphaA, alphaB, rowsA, rowsB, msgA,
                  msgB, agg_sh, gsemA, gsemB, asemA, asemB, ssemA, ssemB):
    cid = lax.axis_index("c")
    sid = lax.axis_index("s")
    wid = cid * NS + sid
    r0 = sid * ROWS_T
    pltpu.sync_copy(lidx_hbm, lidx_v)
    lay = lidx_v[...][0]

    pltpu.sync_copy(zerosm_hbm, agg_sh.at[pl.ds(r0, ROWS_T)])

    pltpu.sync_copy(src2_hbm.at[wid], src2_v)
    pltpu.sync_copy(dst2_hbm.at[wid], dst2_v)

    def compute(rows_v, alpha_v, msg_v):
        @plsc.parallel_loop(0, C, 1, unroll=2)
        def _(e):
            av = alpha_v[e, :]
            a = [av[k] for k in range(KER)]
            for seg in range(OUT // 16):
                col = seg * 16
                acc = a[0] * rows_v[e, pl.ds(col, 16)]
                for k in range(1, KER):
                    acc = acc + a[k] * rows_v[e, pl.ds(k * OUT + col, 16)]
                msg_v[e, pl.ds(col, 16)] = acc

    def start(j, rows_v, alpha_v, gsem, asem):
        pltpu.async_copy(alpha_hbm.at[lay, wid, j], alpha_v, asem)
        pltpu.async_copy(y_hbm.at[src2_v.at[j]], rows_v, gsem)

    def wait(j, rows_v, alpha_v, gsem, asem):
        pltpu.make_async_copy(alpha_hbm.at[lay, wid, j], alpha_v, asem).wait()
        pltpu.make_async_copy(y_hbm.at[src2_v.at[j]], rows_v, gsem).wait()

    start(0, rowsA, alphaA, gsemA, asemA)
    plsc.subcore_barrier()

    def drain(msg_v, ssem):
        pltpu.make_async_copy(msg_v, agg_sh.at[dst2_v.at[0]], ssem).wait()

    def pair(jj, carry):
        j0 = jj * 2
        j1 = j0 + 1
        start(j1, rowsB, alphaB, gsemB, asemB)
        wait(j0, rowsA, alphaA, gsemA, asemA)

        @pl.when(jj > 0)
        def _():
            drain(msgA, ssemA)

        compute(rowsA, alphaA, msgA)
        pltpu.async_copy(msgA, agg_sh.at[dst2_v.at[j0]], ssemA, add=True)

        @pl.when(jj < NCH // 2 - 1)
        def _():
            start(j0 + 2, rowsA, alphaA, gsemA, asemA)

        wait(j1, rowsB, alphaB, gsemB, asemB)

        @pl.when(jj > 0)
        def _():
            drain(msgB, ssemB)

        compute(rowsB, alphaB, msgB)
        pltpu.async_copy(msgB, agg_sh.at[dst2_v.at[j1]], ssemB, add=True)
        return carry

    lax.fori_loop(0, NCH // 2, pair, 0)
    drain(msgA, ssemA)
    drain(msgB, ssemB)
    plsc.subcore_barrier()

    pltpu.sync_copy(agg_sh.at[pl.ds(r0, ROWS_T)],
                    agg_hbm.at[cid, pl.ds(r0, ROWS_T)])


_sc_edge = functools.partial(
    pl.kernel,
    compiler_params=pltpu.CompilerParams(use_tc_tiling_on_sc=False),
    out_type=[jax.ShapeDtypeStruct((NC, NP, OUT), jnp.float32)],
    mesh=_mesh,
    scratch_types=[
        pltpu.VMEM((NCH, C), jnp.int32),
        pltpu.VMEM((NCH, C), jnp.int32),
        pltpu.VMEM((16,), jnp.int32),
        pltpu.VMEM((C, 16), jnp.float32),
        pltpu.VMEM((C, 16), jnp.float32),
        pltpu.VMEM((C, KER * OUT), jnp.float32),
        pltpu.VMEM((C, KER * OUT), jnp.float32),
        pltpu.VMEM((C, OUT), jnp.float32),
        pltpu.VMEM((C, OUT), jnp.float32),
        pltpu.VMEM_SHARED((NP, OUT), jnp.float32),
        pltpu.SemaphoreType.DMA,
        pltpu.SemaphoreType.DMA,
        pltpu.SemaphoreType.DMA,
        pltpu.SemaphoreType.DMA,
        pltpu.SemaphoreType.DMA,
        pltpu.SemaphoreType.DMA,
    ],
)(_sc_edge_body)


_EB = 1280


def _tc_alpha_body(p_ref, w1_ref, b1_ref, w2_ref, b2_ref, a_ref):
    p = p_ref[0]
    for l in range(4):
        h = jnp.tanh(
            jnp.dot(w1_ref[l], p, preferred_element_type=jnp.float32)
            + b1_ref[l])
        t = (jnp.dot(w2_ref[l], h, preferred_element_type=jnp.float32)
             + b2_ref[l])
        t = t - jnp.max(t, axis=0, keepdims=True)
        ex = jnp.exp(t)
        a8 = ex / jnp.sum(ex, axis=0, keepdims=True)
        a16 = jnp.concatenate([a8, jnp.zeros_like(a8)], axis=0)
        a_ref[l, 0] = jnp.transpose(a16)


_BPW = EPWP // _EB


def _tc_alpha(pseudo, w1, b1, w2, b2):
    return pl.pallas_call(
        _tc_alpha_body,
        grid=(NW * _BPW,),
        in_specs=[
            pl.BlockSpec((1, 3, _EB), lambda i: (i // _BPW, 0, i % _BPW)),
            pl.BlockSpec((4, 16, 3), lambda i: (0, 0, 0)),
            pl.BlockSpec((4, 16, 1), lambda i: (0, 0, 0)),
            pl.BlockSpec((4, 8, 16), lambda i: (0, 0, 0)),
            pl.BlockSpec((4, 8, 1), lambda i: (0, 0, 0)),
        ],
        out_specs=pl.BlockSpec((4, 1, _EB, 16),
                               lambda i: (0, i // _BPW, i % _BPW, 0)),
        out_shape=jax.ShapeDtypeStruct((4, NW, EPWP, 16), jnp.float32),
    )(pseudo, w1, b1, w2, b2)


_MB = 400


def _tc_mm_body(x_ref, w_ref, y_ref):
    y_ref[...] = jnp.dot(x_ref[...], w_ref[...],
                         preferred_element_type=jnp.float32)


def _tc_mm(feats, w):
    k = feats.shape[1]
    return pl.pallas_call(
        _tc_mm_body,
        grid=(N // _MB,),
        in_specs=[
            pl.BlockSpec((_MB, k), lambda i: (i, 0)),
            pl.BlockSpec((k, KER * OUT), lambda i: (0, 0)),
        ],
        out_specs=pl.BlockSpec((_MB, KER * OUT), lambda i: (i, 0)),
        out_shape=jax.ShapeDtypeStruct((N, KER * OUT), jnp.float32),
    )(feats, w)


def _tc_mm2_body(b_ref, x_ref, w_ref, y_ref):
    y_ref[...] = b_ref[...] + jnp.dot(x_ref[...], w_ref[...],
                                      preferred_element_type=jnp.float32)


def _tc_mm2(base, xl, w):
    return pl.pallas_call(
        _tc_mm2_body,
        grid=(N // _MB,),
        in_specs=[
            pl.BlockSpec((_MB, KER * OUT), lambda i: (i, 0)),
            pl.BlockSpec((_MB, OUT), lambda i: (i, 0)),
            pl.BlockSpec((OUT, KER * OUT), lambda i: (0, 0)),
        ],
        out_specs=pl.BlockSpec((_MB, KER * OUT), lambda i: (i, 0)),
        out_shape=jax.ShapeDtypeStruct((N, KER * OUT), jnp.float32),
    )(base, xl, w)


def _leaky(v):
    return jnp.where(v >= 0, v, 0.01 * v)


def _tc_act_body(agg_ref, deg_ref, b_ref, x_ref):
    agg = agg_ref[0] + agg_ref[1]
    deg = deg_ref[0] + deg_ref[1]
    v = agg / jnp.maximum(deg, 1.0) + b_ref[...]
    x_ref[...] = _leaky(v)


def _tc_act(aggp, degp, b):
    return pl.pallas_call(
        _tc_act_body,
        grid=(N // _MB,),
        in_specs=[
            pl.BlockSpec((NC, _MB, OUT), lambda i: (0, i, 0)),
            pl.BlockSpec((NC, _MB, 1), lambda i: (0, i, 0)),
            pl.BlockSpec((1, OUT), lambda i: (0, 0)),
        ],
        out_specs=pl.BlockSpec((_MB, OUT), lambda i: (i, 0)),
        out_shape=jax.ShapeDtypeStruct((N, OUT), jnp.float32),
    )(aggp, degp, b)


def _tc_pool_body(x4_ref, w1_ref, c1_ref, w2_ref, c2_ref,
                  w3_ref, c3_ref, z_ref, acc):
    i = pl.program_id(0)

    @pl.when(i == 0)
    def _():
        acc[...] = jnp.zeros_like(acc)

    acc[...] += jnp.sum(x4_ref[...], axis=0, keepdims=True)

    @pl.when(i == N // _MB - 1)
    def _():
        pooled = acc[...] * (1.0 / N)
        z1 = _leaky(jnp.dot(pooled, w1_ref[...],
                            preferred_element_type=jnp.float32) + c1_ref[...])
        z2 = _leaky(jnp.dot(z1, w2_ref[...],
                            preferred_element_type=jnp.float32) + c2_ref[...])
        z_ref[...] = (jnp.dot(z2, w3_ref[...],
                              preferred_element_type=jnp.float32)
                      + c3_ref[...])


def _tc_pool(x4, w1, c1, w2, c2, w3, c3):
    full = lambda *shape: pl.BlockSpec(shape, lambda i: (0,) * len(shape))
    return pl.pallas_call(
        _tc_pool_body,
        grid=(N // _MB,),
        in_specs=[
            pl.BlockSpec((_MB, OUT), lambda i: (i, 0)),
            full(OUT, 16), full(1, 16),
            full(16, 8), full(1, 8),
            full(8, 1), full(1, 1),
        ],
        out_specs=full(1, 1),
        out_shape=jax.ShapeDtypeStruct((1, 1), jnp.float32),
        scratch_shapes=[pltpu.VMEM((1, OUT), jnp.float32)],
    )(x4, w1, c1, w2, c2, w3, c3)


FIN = 320


def kernel(x, edge_index, params):
    src = edge_index[0]
    dst = edge_index[1]

    src2 = src.reshape(NW, NCH, C)
    dst2 = dst.reshape(NW, NCH, C)
    src2p = src.reshape(NW, NCHP, CP)
    dst2p = dst.reshape(NW, NCHP, CP)
    zeros1 = jnp.zeros((ROWS_T,), jnp.float32)
    zerosm = jnp.zeros((ROWS_T, OUT), jnp.float32)

    spec_flat = x[:, :3].reshape(-1)
    pseudo_w, degp = _sc_prep(spec_flat, src2p, dst2p, zeros1)
    degp3 = degp.reshape(NC, NP, 1)

    gcs = [params["gc%d" % (i + 1)] for i in range(4)]
    w1 = jnp.stack([g["Kw1"].T for g in gcs])
    b1 = jnp.stack([g["Kb1"].reshape(16, 1) for g in gcs])
    w2 = jnp.stack([g["Kw2"].T for g in gcs])
    b2 = jnp.stack([g["Kb2"].reshape(8, 1) for g in gcs])
    alpha = _tc_alpha(pseudo_w, w1, b1, w2, b2)
    al_all = alpha.reshape(4, NW, NCHA, C, 16)

    zblk = jnp.zeros((KER, OUT, OUT), jnp.float32)
    wflat = []
    for l in range(4):
        wk = gcs[l]["Wk"]
        parts = [wk[:, l * OUT:l * OUT + 128]]
        for m in range(1, 4):
            j = l - m
            parts.append(wk[:, j * OUT:(j + 1) * OUT] if 0 <= j <= l - 1
                         else zblk)
        wpad = jnp.concatenate(parts, axis=1)
        wflat.append(jnp.transpose(wpad, (1, 0, 2)).reshape(FIN, KER * OUT))
    wflat_all = jnp.stack(wflat)
    bias_all = jnp.stack([g["b"].reshape(1, OUT) for g in gcs])

    feats0 = jnp.concatenate(
        [x, jnp.zeros((N, FIN - 128), jnp.float32)], axis=1)

    lidx_all = jnp.broadcast_to(
        jnp.arange(4, dtype=jnp.int32)[:, None], (4, 16))

    wflat_next = jnp.concatenate([wflat_all[1:], wflat_all[:1]])
    wnew_all = jnp.stack(
        [wflat_next[l, 128 + min(l, 2) * OUT:128 + min(l, 2) * OUT + OUT]
         for l in range(4)])

    y0 = _tc_mm(feats0, wflat_all[0])

    def step(carry, xs_l):
        feats, y = carry
        wnext_l, wnew_l, bias_l, lidx_l, l_idx = xs_l
        aggp, = _sc_edge(y, src2, dst2, al_all, lidx_l, zerosm)
        ybase = _tc_mm(feats, wnext_l)
        xl = _tc_act(aggp, degp3, bias_l)
        off = 128 + jnp.minimum(l_idx, 2) * OUT
        feats = lax.dynamic_update_slice(feats, xl, (0, off))
        y_next = _tc_mm2(ybase, xl, wnew_l)
        return (feats, y_next), xl

    _, xls = lax.scan(step, (feats0, y0),
                      (wflat_next, wnew_all, bias_all, lidx_all,
                       jnp.arange(4)))
    x4 = xls[3]

    z = _tc_pool(x4,
                 params["lin1_w"], params["lin1_b"].reshape(1, 16),
                 params["lin2_w"], params["lin2_b"].reshape(1, 8),
                 params["lin3_w"], params["lin3_b"].reshape(1, 1))
    return z.reshape(1)


# speedup vs baseline: 1.0237x; 1.0237x over previous
import functools

import jax
import jax.numpy as jnp
from jax import lax
from jax.experimental import pallas as pl
from jax.experimental.pallas import tpu as pltpu
from jax.experimental.pallas import tpu_sc as plsc

N = 10000
E = 320000
KER = 8
OUT = 64
NC = 2
NS = 16
NW = NC * NS
EPW = E // NW
C = 40
NCH = EPW // C
CP = 80
NCHP = EPW // CP
NP = 10240
ROWS_T = NP // NS
EPWP = 10240
NCHA = EPWP // C

_mesh = plsc.VectorSubcoreMesh(
    core_axis_name="c", subcore_axis_name="s", num_cores=NC, num_subcores=NS)


def _sc_prep_body(spec_hbm, src2p_hbm, dst2p_hbm, zeros1_hbm,
                  pseudo_hbm, deg_hbm,
                  spec_v, src1_v, dst1_v, pbuf_v, ones_v, deg_sh,
                  dsem):
    cid = lax.axis_index("c")
    sid = lax.axis_index("s")
    wid = cid * NS + sid

    r0 = sid * ROWS_T
    pltpu.sync_copy(zeros1_hbm, deg_sh.at[pl.ds(r0, ROWS_T)])

    pltpu.sync_copy(src2p_hbm.at[wid], src1_v)
    pltpu.sync_copy(dst2p_hbm.at[wid], dst1_v)
    pltpu.sync_copy(spec_hbm, spec_v)

    for i in range(CP // 16):
        ones_v[pl.ds(i * 16, 16)] = jnp.ones((16,), jnp.float32)

    plsc.subcore_barrier()

    def group(g, carry):
        r = g // (CP // 16)
        c0 = (g % (CP // 16)) * 16
        sv = src1_v[r, pl.ds(c0, 16)] * 3
        dv = dst1_v[r, pl.ds(c0, 16)] * 3
        for c in range(3):
            cc = jnp.full((16,), c, jnp.int32)
            ss = plsc.load_gather(spec_v, [sv + cc])
            sd = plsc.load_gather(spec_v, [dv + cc])
            pbuf_v[c, pl.ds(g * 16, 16)] = sd - ss
        return carry

    lax.fori_loop(0, EPW // 16, group, 0)
    pltpu.sync_copy(pbuf_v, pseudo_hbm.at[wid, :, pl.ds(0, EPW)])

    def dchunk(j, carry):
        pltpu.async_copy(ones_v, deg_sh.at[dst1_v.at[j]], dsem, add=True)

        @pl.when(j >= 8)
        def _():
            pltpu.make_async_copy(ones_v, deg_sh.at[dst1_v.at[0]],
                                  dsem).wait()

        return carry

    lax.fori_loop(0, NCHP, dchunk, 0)

    def ddrain(j, carry):
        pltpu.make_async_copy(ones_v, deg_sh.at[dst1_v.at[0]], dsem).wait()
        return carry

    lax.fori_loop(0, 8, ddrain, 0)
    plsc.subcore_barrier()

    pltpu.sync_copy(deg_sh.at[pl.ds(r0, ROWS_T)],
                    deg_hbm.at[cid, 0, pl.ds(r0, ROWS_T)])


_sc_prep = functools.partial(
    pl.kernel,
    compiler_params=pltpu.CompilerParams(needs_layout_passes=False,
                                         use_tc_tiling_on_sc=False),
    out_type=[jax.ShapeDtypeStruct((NW, 3, EPWP), jnp.float32),
              jax.ShapeDtypeStruct((NC, 1, NP), jnp.float32)],
    mesh=_mesh,
    scratch_types=[
        pltpu.VMEM((3 * N,), jnp.float32),
        pltpu.VMEM((NCHP, CP), jnp.int32),
        pltpu.VMEM((NCHP, CP), jnp.int32),
        pltpu.VMEM((3, EPW), jnp.float32),
        pltpu.VMEM((CP,), jnp.float32),
        pltpu.VMEM_SHARED((NP,), jnp.float32),
        pltpu.SemaphoreType.DMA,
    ],
)(_sc_prep_body)


def _sc_edge_body(y_hbm, src2_hbm, dst2_hbm, alpha_hbm, lidx_hbm, zerosm_hbm,
                  agg_hbm,
                  src2_v, dst2_v, lidx_v, alphaA, alphaB, rowsA, rowsB, msgA,
                  msgB, agg_sh, gsemA, gsemB, asemA, asemB, ssemA, ssemB):
    cid = lax.axis_index("c")
    sid = lax.axis_index("s")
    wid = cid * NS + sid
    r0 = sid * ROWS_T
    pltpu.sync_copy(lidx_hbm, lidx_v)
    lay = lidx_v[...][0]

    pltpu.sync_copy(zerosm_hbm, agg_sh.at[pl.ds(r0, ROWS_T)])

    pltpu.sync_copy(src2_hbm.at[wid], src2_v)
    pltpu.sync_copy(dst2_hbm.at[wid], dst2_v)

    def compute(rows_v, alpha_v, msg_v):
        @plsc.parallel_loop(0, C, 1, unroll=2)
        def _(e):
            av = alpha_v[e, :]
            a = [av[k] for k in range(KER)]
            for seg in range(OUT // 16):
                col = seg * 16
                acc = a[0] * rows_v[e, pl.ds(col, 16)]
                for k in range(1, KER):
                    acc = acc + a[k] * rows_v[e, pl.ds(k * OUT + col, 16)]
                msg_v[e, pl.ds(col, 16)] = acc

    def start(j, rows_v, alpha_v, gsem, asem):
        pltpu.async_copy(alpha_hbm.at[lay, wid, j], alpha_v, asem)
        pltpu.async_copy(y_hbm.at[src2_v.at[j]], rows_v, gsem)

    def wait(j, rows_v, alpha_v, gsem, asem):
        pltpu.make_async_copy(alpha_hbm.at[lay, wid, j], alpha_v, asem).wait()
        pltpu.make_async_copy(y_hbm.at[src2_v.at[j]], rows_v, gsem).wait()

    start(0, rowsA, alphaA, gsemA, asemA)
    plsc.subcore_barrier()

    def drain(msg_v, ssem):
        pltpu.make_async_copy(msg_v, agg_sh.at[dst2_v.at[0]], ssem).wait()

    def pair(jj, carry):
        j0 = jj * 2
        j1 = j0 + 1
        start(j1, rowsB, alphaB, gsemB, asemB)
        wait(j0, rowsA, alphaA, gsemA, asemA)

        @pl.when(jj > 0)
        def _():
            drain(msgA, ssemA)

        compute(rowsA, alphaA, msgA)
        pltpu.async_copy(msgA, agg_sh.at[dst2_v.at[j0]], ssemA, add=True)

        @pl.when(jj < NCH // 2 - 1)
        def _():
            start(j0 + 2, rowsA, alphaA, gsemA, asemA)

        wait(j1, rowsB, alphaB, gsemB, asemB)

        @pl.when(jj > 0)
        def _():
            drain(msgB, ssemB)

        compute(rowsB, alphaB, msgB)
        pltpu.async_copy(msgB, agg_sh.at[dst2_v.at[j1]], ssemB, add=True)
        return carry

    lax.fori_loop(0, NCH // 2, pair, 0)
    drain(msgA, ssemA)
    drain(msgB, ssemB)
    plsc.subcore_barrier()

    pltpu.sync_copy(agg_sh.at[pl.ds(r0, ROWS_T)],
                    agg_hbm.at[cid, pl.ds(r0, ROWS_T)])


_sc_edge = functools.partial(
    pl.kernel,
    compiler_params=pltpu.CompilerParams(use_tc_tiling_on_sc=False),
    out_type=[jax.ShapeDtypeStruct((NC, NP, OUT), jnp.float32)],
    mesh=_mesh,
    scratch_types=[
        pltpu.VMEM((NCH, C), jnp.int32),
        pltpu.VMEM((NCH, C), jnp.int32),
        pltpu.VMEM((16,), jnp.int32),
        pltpu.VMEM((C, 16), jnp.float32),
        pltpu.VMEM((C, 16), jnp.float32),
        pltpu.VMEM((C, KER * OUT), jnp.float32),
        pltpu.VMEM((C, KER * OUT), jnp.float32),
        pltpu.VMEM((C, OUT), jnp.float32),
        pltpu.VMEM((C, OUT), jnp.float32),
        pltpu.VMEM_SHARED((NP, OUT), jnp.float32),
        pltpu.SemaphoreType.DMA,
        pltpu.SemaphoreType.DMA,
        pltpu.SemaphoreType.DMA,
        pltpu.SemaphoreType.DMA,
        pltpu.SemaphoreType.DMA,
        pltpu.SemaphoreType.DMA,
    ],
)(_sc_edge_body)


_EB = 1280


def _tc_alpha_body(p_ref, w1_ref, b1_ref, w2_ref, b2_ref, a_ref):
    p = p_ref[0]
    for l in range(4):
        h = jnp.tanh(
            jnp.dot(w1_ref[l], p, preferred_element_type=jnp.float32)
            + b1_ref[l])
        t = (jnp.dot(w2_ref[l], h, preferred_element_type=jnp.float32)
             + b2_ref[l])
        t = t - jnp.max(t, axis=0, keepdims=True)
        ex = jnp.exp(t)
        a8 = ex / jnp.sum(ex, axis=0, keepdims=True)
        a16 = jnp.concatenate([a8, jnp.zeros_like(a8)], axis=0)
        a_ref[l, 0] = jnp.transpose(a16)


_BPW = EPWP // _EB


def _tc_alpha(pseudo, w1, b1, w2, b2):
    return pl.pallas_call(
        _tc_alpha_body,
        grid=(NW * _BPW,),
        in_specs=[
            pl.BlockSpec((1, 3, _EB), lambda i: (i // _BPW, 0, i % _BPW)),
            pl.BlockSpec((4, 16, 3), lambda i: (0, 0, 0)),
            pl.BlockSpec((4, 16, 1), lambda i: (0, 0, 0)),
            pl.BlockSpec((4, 8, 16), lambda i: (0, 0, 0)),
            pl.BlockSpec((4, 8, 1), lambda i: (0, 0, 0)),
        ],
        out_specs=pl.BlockSpec((4, 1, _EB, 16),
                               lambda i: (0, i // _BPW, i % _BPW, 0)),
        out_shape=jax.ShapeDtypeStruct((4, NW, EPWP, 16), jnp.float32),
    )(pseudo, w1, b1, w2, b2)


_MB = 400


def _tc_mm_body(x_ref, w_ref, y_ref):
    y_ref[...] = jnp.dot(x_ref[...], w_ref[...],
                         preferred_element_type=jnp.float32)


def _tc_mm(feats, w):
    k = feats.shape[1]
    return pl.pallas_call(
        _tc_mm_body,
        grid=(N // _MB,),
        in_specs=[
            pl.BlockSpec((_MB, k), lambda i: (i, 0)),
            pl.BlockSpec((k, KER * OUT), lambda i: (0, 0)),
        ],
        out_specs=pl.BlockSpec((_MB, KER * OUT), lambda i: (i, 0)),
        out_shape=jax.ShapeDtypeStruct((N, KER * OUT), jnp.float32),
    )(feats, w)


def _leaky(v):
    return jnp.where(v >= 0, v, 0.01 * v)


def _tc_act_body(agg_ref, deg_ref, b_ref, x_ref):
    agg = agg_ref[0] + agg_ref[1]
    deg = deg_ref[0] + deg_ref[1]
    v = agg / jnp.maximum(deg, 1.0) + b_ref[...]
    x_ref[...] = _leaky(v)


def _tc_act(aggp, degp, b):
    return pl.pallas_call(
        _tc_act_body,
        grid=(N // _MB,),
        in_specs=[
            pl.BlockSpec((NC, _MB, OUT), lambda i: (0, i, 0)),
            pl.BlockSpec((NC, _MB, 1), lambda i: (0, i, 0)),
            pl.BlockSpec((1, OUT), lambda i: (0, 0)),
        ],
        out_specs=pl.BlockSpec((_MB, OUT), lambda i: (i, 0)),
        out_shape=jax.ShapeDtypeStruct((N, OUT), jnp.float32),
    )(aggp, degp, b)


def _tc_pool_body(x4_ref, w1_ref, c1_ref, w2_ref, c2_ref,
                  w3_ref, c3_ref, z_ref, acc):
    i = pl.program_id(0)

    @pl.when(i == 0)
    def _():
        acc[...] = jnp.zeros_like(acc)

    acc[...] += jnp.sum(x4_ref[...], axis=0, keepdims=True)

    @pl.when(i == N // _MB - 1)
    def _():
        pooled = acc[...] * (1.0 / N)
        z1 = _leaky(jnp.dot(pooled, w1_ref[...],
                            preferred_element_type=jnp.float32) + c1_ref[...])
        z2 = _leaky(jnp.dot(z1, w2_ref[...],
                            preferred_element_type=jnp.float32) + c2_ref[...])
        z_ref[...] = (jnp.dot(z2, w3_ref[...],
                              preferred_element_type=jnp.float32)
                      + c3_ref[...])


def _tc_pool(x4, w1, c1, w2, c2, w3, c3):
    full = lambda *shape: pl.BlockSpec(shape, lambda i: (0,) * len(shape))
    return pl.pallas_call(
        _tc_pool_body,
        grid=(N // _MB,),
        in_specs=[
            pl.BlockSpec((_MB, OUT), lambda i: (i, 0)),
            full(OUT, 16), full(1, 16),
            full(16, 8), full(1, 8),
            full(8, 1), full(1, 1),
        ],
        out_specs=full(1, 1),
        out_shape=jax.ShapeDtypeStruct((1, 1), jnp.float32),
        scratch_shapes=[pltpu.VMEM((1, OUT), jnp.float32)],
    )(x4, w1, c1, w2, c2, w3, c3)


FIN = 320


def kernel(x, edge_index, params):
    src = edge_index[0]
    dst = edge_index[1]

    src2 = src.reshape(NW, NCH, C)
    dst2 = dst.reshape(NW, NCH, C)
    src2p = src.reshape(NW, NCHP, CP)
    dst2p = dst.reshape(NW, NCHP, CP)
    zeros1 = jnp.zeros((ROWS_T,), jnp.float32)
    zerosm = jnp.zeros((ROWS_T, OUT), jnp.float32)

    spec_flat = x[:, :3].reshape(-1)
    pseudo_w, degp = _sc_prep(spec_flat, src2p, dst2p, zeros1)
    degp3 = degp.reshape(NC, NP, 1)

    gcs = [params["gc%d" % (i + 1)] for i in range(4)]
    w1 = jnp.stack([g["Kw1"].T for g in gcs])
    b1 = jnp.stack([g["Kb1"].reshape(16, 1) for g in gcs])
    w2 = jnp.stack([g["Kw2"].T for g in gcs])
    b2 = jnp.stack([g["Kb2"].reshape(8, 1) for g in gcs])
    alpha = _tc_alpha(pseudo_w, w1, b1, w2, b2)
    al_all = alpha.reshape(4, NW, NCHA, C, 16)

    zblk = jnp.zeros((KER, OUT, OUT), jnp.float32)
    wflat = []
    for l in range(4):
        wk = gcs[l]["Wk"]
        parts = [wk[:, l * OUT:l * OUT + 128]]
        for m in range(1, 4):
            j = l - m
            parts.append(wk[:, j * OUT:(j + 1) * OUT] if 0 <= j <= l - 1
                         else zblk)
        wpad = jnp.concatenate(parts, axis=1)
        wflat.append(jnp.transpose(wpad, (1, 0, 2)).reshape(FIN, KER * OUT))
    wflat_all = jnp.stack(wflat)
    bias_all = jnp.stack([g["b"].reshape(1, OUT) for g in gcs])

    feats0 = jnp.concatenate(
        [x, jnp.zeros((N, FIN - 128), jnp.float32)], axis=1)

    lidx_all = jnp.broadcast_to(
        jnp.arange(4, dtype=jnp.int32)[:, None], (4, 16))

    def step(feats, xs_l):
        wflat_l, bias_l, lidx_l, l_idx = xs_l
        y = _tc_mm(feats, wflat_l)
        aggp, = _sc_edge(y, src2, dst2, al_all, lidx_l, zerosm)
        xl = _tc_act(aggp, degp3, bias_l)
        off = 128 + jnp.minimum(l_idx, 2) * OUT
        feats = lax.dynamic_update_slice(feats, xl, (0, off))
        return feats, xl

    _, xls = lax.scan(step, feats0,
                      (wflat_all, bias_all, lidx_all, jnp.arange(4)))
    x4 = xls[3]

    z = _tc_pool(x4,
                 params["lin1_w"], params["lin1_b"].reshape(1, 16),
                 params["lin2_w"], params["lin2_b"].reshape(1, 8),
                 params["lin3_w"], params["lin3_b"].reshape(1, 1))
    return z.reshape(1)


# speedup vs baseline: 1.1311x; 1.1049x over previous
import functools

import jax
import jax.numpy as jnp
from jax import lax
from jax.experimental import pallas as pl
from jax.experimental.pallas import tpu as pltpu
from jax.experimental.pallas import tpu_sc as plsc

N = 10000
E = 320000
KER = 8
OUT = 64
NC = 2
NS = 16
NW = NC * NS
EPW = E // NW
C = 40
NCH = EPW // C
CP = 80
NCHP = EPW // CP
NP = 10240
ROWS_T = NP // NS
EPWP = 10240
NCHA = EPWP // C

_mesh = plsc.VectorSubcoreMesh(
    core_axis_name="c", subcore_axis_name="s", num_cores=NC, num_subcores=NS)


def _sc_prep_body(spec_hbm, src2p_hbm, dst2p_hbm, zeros1_hbm,
                  pseudo_hbm, deg_hbm,
                  spec_v, src1_v, dst1_v, pbuf_v, ones_v, deg_sh,
                  dsem):
    cid = lax.axis_index("c")
    sid = lax.axis_index("s")
    wid = cid * NS + sid

    r0 = sid * ROWS_T
    pltpu.sync_copy(zeros1_hbm, deg_sh.at[pl.ds(r0, ROWS_T)])

    pltpu.sync_copy(src2p_hbm.at[wid], src1_v)
    pltpu.sync_copy(dst2p_hbm.at[wid], dst1_v)
    pltpu.sync_copy(spec_hbm, spec_v)

    for i in range(CP // 16):
        ones_v[pl.ds(i * 16, 16)] = jnp.ones((16,), jnp.float32)

    plsc.subcore_barrier()

    def group(g, carry):
        r = g // (CP // 16)
        c0 = (g % (CP // 16)) * 16
        sv = src1_v[r, pl.ds(c0, 16)] * 3
        dv = dst1_v[r, pl.ds(c0, 16)] * 3
        for c in range(3):
            cc = jnp.full((16,), c, jnp.int32)
            ss = plsc.load_gather(spec_v, [sv + cc])
            sd = plsc.load_gather(spec_v, [dv + cc])
            pbuf_v[c, pl.ds(g * 16, 16)] = sd - ss
        return carry

    lax.fori_loop(0, EPW // 16, group, 0)
    pltpu.sync_copy(pbuf_v, pseudo_hbm.at[wid, :, pl.ds(0, EPW)])

    def dchunk(j, carry):
        pltpu.async_copy(ones_v, deg_sh.at[dst1_v.at[j]], dsem, add=True)

        @pl.when(j >= 8)
        def _():
            pltpu.make_async_copy(ones_v, deg_sh.at[dst1_v.at[0]],
                                  dsem).wait()

        return carry

    lax.fori_loop(0, NCHP, dchunk, 0)

    def ddrain(j, carry):
        pltpu.make_async_copy(ones_v, deg_sh.at[dst1_v.at[0]], dsem).wait()
        return carry

    lax.fori_loop(0, 8, ddrain, 0)
    plsc.subcore_barrier()

    pltpu.sync_copy(deg_sh.at[pl.ds(r0, ROWS_T)],
                    deg_hbm.at[cid, 0, pl.ds(r0, ROWS_T)])


_sc_prep = functools.partial(
    pl.kernel,
    compiler_params=pltpu.CompilerParams(needs_layout_passes=False,
                                         use_tc_tiling_on_sc=False),
    out_type=[jax.ShapeDtypeStruct((NW, 3, EPWP), jnp.float32),
              jax.ShapeDtypeStruct((NC, 1, NP), jnp.float32)],
    mesh=_mesh,
    scratch_types=[
        pltpu.VMEM((3 * N,), jnp.float32),
        pltpu.VMEM((NCHP, CP), jnp.int32),
        pltpu.VMEM((NCHP, CP), jnp.int32),
        pltpu.VMEM((3, EPW), jnp.float32),
        pltpu.VMEM((CP,), jnp.float32),
        pltpu.VMEM_SHARED((NP,), jnp.float32),
        pltpu.SemaphoreType.DMA,
    ],
)(_sc_prep_body)


def _sc_edge_body(y_hbm, src2_hbm, dst2_hbm, alpha_hbm, lidx_hbm, zerosm_hbm,
                  agg_hbm,
                  src2_v, dst2_v, lidx_v, alphaA, alphaB, rowsA, rowsB, msgA,
                  msgB, agg_sh, gsemA, gsemB, asemA, asemB, ssemA, ssemB):
    cid = lax.axis_index("c")
    sid = lax.axis_index("s")
    wid = cid * NS + sid
    r0 = sid * ROWS_T
    pltpu.sync_copy(lidx_hbm, lidx_v)
    lay = lidx_v[...][0]

    pltpu.sync_copy(zerosm_hbm, agg_sh.at[pl.ds(r0, ROWS_T)])

    pltpu.sync_copy(src2_hbm.at[wid], src2_v)
    pltpu.sync_copy(dst2_hbm.at[wid], dst2_v)

    iot16 = lax.iota(jnp.int32, 16)

    def compute(rows_v, alpha_v, msg_v):
        @plsc.parallel_loop(0, C, 1, unroll=2)
        def _(e):
            ecol = iot16 * 0 + e
            av = plsc.load_gather(alpha_v, [iot16, ecol])
            a = [av[k] for k in range(KER)]
            for seg in range(OUT // 16):
                col = seg * 16
                acc = a[0] * rows_v[e, pl.ds(col, 16)]
                for k in range(1, KER):
                    acc = acc + a[k] * rows_v[e, pl.ds(k * OUT + col, 16)]
                msg_v[e, pl.ds(col, 16)] = acc

    def start(j, rows_v, alpha_v, gsem, asem):
        pltpu.async_copy(alpha_hbm.at[lay, wid, :, pl.ds(j * C, C)],
                         alpha_v, asem)
        pltpu.async_copy(y_hbm.at[src2_v.at[j]], rows_v, gsem)

    def wait(j, rows_v, alpha_v, gsem, asem):
        pltpu.make_async_copy(alpha_hbm.at[lay, wid, :, pl.ds(j * C, C)],
                              alpha_v, asem).wait()
        pltpu.make_async_copy(y_hbm.at[src2_v.at[j]], rows_v, gsem).wait()

    start(0, rowsA, alphaA, gsemA, asemA)
    plsc.subcore_barrier()

    def drain(msg_v, ssem):
        pltpu.make_async_copy(msg_v, agg_sh.at[dst2_v.at[0]], ssem).wait()

    def pair(jj, carry):
        j0 = jj * 2
        j1 = j0 + 1
        start(j1, rowsB, alphaB, gsemB, asemB)
        wait(j0, rowsA, alphaA, gsemA, asemA)

        @pl.when(jj > 0)
        def _():
            drain(msgA, ssemA)

        compute(rowsA, alphaA, msgA)
        pltpu.async_copy(msgA, agg_sh.at[dst2_v.at[j0]], ssemA, add=True)

        @pl.when(jj < NCH // 2 - 1)
        def _():
            start(j0 + 2, rowsA, alphaA, gsemA, asemA)

        wait(j1, rowsB, alphaB, gsemB, asemB)

        @pl.when(jj > 0)
        def _():
            drain(msgB, ssemB)

        compute(rowsB, alphaB, msgB)
        pltpu.async_copy(msgB, agg_sh.at[dst2_v.at[j1]], ssemB, add=True)
        return carry

    lax.fori_loop(0, NCH // 2, pair, 0)
    drain(msgA, ssemA)
    drain(msgB, ssemB)
    plsc.subcore_barrier()

    pltpu.sync_copy(agg_sh.at[pl.ds(r0, ROWS_T)],
                    agg_hbm.at[cid, pl.ds(r0, ROWS_T)])


_sc_edge = functools.partial(
    pl.kernel,
    compiler_params=pltpu.CompilerParams(use_tc_tiling_on_sc=False,
                                         needs_layout_passes=False),
    out_type=[jax.ShapeDtypeStruct((NC, NP, OUT), jnp.float32)],
    mesh=_mesh,
    scratch_types=[
        pltpu.VMEM((NCH, C), jnp.int32),
        pltpu.VMEM((NCH, C), jnp.int32),
        pltpu.VMEM((16,), jnp.int32),
        pltpu.VMEM((16, C), jnp.float32),
        pltpu.VMEM((16, C), jnp.float32),
        pltpu.VMEM((C, KER * OUT), jnp.float32),
        pltpu.VMEM((C, KER * OUT), jnp.float32),
        pltpu.VMEM((C, OUT), jnp.float32),
        pltpu.VMEM((C, OUT), jnp.float32),
        pltpu.VMEM_SHARED((NP, OUT), jnp.float32),
        pltpu.SemaphoreType.DMA,
        pltpu.SemaphoreType.DMA,
        pltpu.SemaphoreType.DMA,
        pltpu.SemaphoreType.DMA,
        pltpu.SemaphoreType.DMA,
        pltpu.SemaphoreType.DMA,
    ],
)(_sc_edge_body)


_EB = 1280


def _tc_alpha_body(p_ref, w1_ref, b1_ref, w2_ref, b2_ref, a_ref):
    p = p_ref[0]
    for l in range(4):
        h = jnp.tanh(
            jnp.dot(w1_ref[l], p, preferred_element_type=jnp.float32)
            + b1_ref[l])
        t = (jnp.dot(w2_ref[l], h, preferred_element_type=jnp.float32)
             + b2_ref[l])
        t = t - jnp.max(t, axis=0, keepdims=True)
        ex = jnp.exp(t)
        a8 = ex / jnp.sum(ex, axis=0, keepdims=True)
        a_ref[l, 0] = jnp.concatenate([a8, jnp.zeros_like(a8)], axis=0)


_BPW = EPWP // _EB


def _tc_alpha(pseudo, w1, b1, w2, b2):
    return pl.pallas_call(
        _tc_alpha_body,
        grid=(NW * _BPW,),
        in_specs=[
            pl.BlockSpec((1, 3, _EB), lambda i: (i // _BPW, 0, i % _BPW)),
            pl.BlockSpec((4, 16, 3), lambda i: (0, 0, 0)),
            pl.BlockSpec((4, 16, 1), lambda i: (0, 0, 0)),
            pl.BlockSpec((4, 8, 16), lambda i: (0, 0, 0)),
            pl.BlockSpec((4, 8, 1), lambda i: (0, 0, 0)),
        ],
        out_specs=pl.BlockSpec((4, 1, 16, _EB),
                               lambda i: (0, i // _BPW, 0, i % _BPW)),
        out_shape=jax.ShapeDtypeStruct((4, NW, 16, EPWP), jnp.float32),
    )(pseudo, w1, b1, w2, b2)


_MB = 400


def _tc_mm_body(x_ref, w_ref, y_ref):
    y_ref[...] = jnp.dot(x_ref[...], w_ref[...],
                         preferred_element_type=jnp.float32)


def _tc_mm(feats, w):
    k = feats.shape[1]
    return pl.pallas_call(
        _tc_mm_body,
        grid=(N // _MB,),
        in_specs=[
            pl.BlockSpec((_MB, k), lambda i: (i, 0)),
            pl.BlockSpec((k, KER * OUT), lambda i: (0, 0)),
        ],
        out_specs=pl.BlockSpec((_MB, KER * OUT), lambda i: (i, 0)),
        out_shape=jax.ShapeDtypeStruct((N, KER * OUT), jnp.float32),
    )(feats, w)


def _leaky(v):
    return jnp.where(v >= 0, v, 0.01 * v)


def _tc_act_body(agg_ref, deg_ref, b_ref, x_ref):
    agg = agg_ref[0] + agg_ref[1]
    deg = deg_ref[0] + deg_ref[1]
    v = agg / jnp.maximum(deg, 1.0) + b_ref[...]
    x_ref[...] = _leaky(v)


def _tc_act(aggp, degp, b):
    return pl.pallas_call(
        _tc_act_body,
        grid=(N // _MB,),
        in_specs=[
            pl.BlockSpec((NC, _MB, OUT), lambda i: (0, i, 0)),
            pl.BlockSpec((NC, _MB, 1), lambda i: (0, i, 0)),
            pl.BlockSpec((1, OUT), lambda i: (0, 0)),
        ],
        out_specs=pl.BlockSpec((_MB, OUT), lambda i: (i, 0)),
        out_shape=jax.ShapeDtypeStruct((N, OUT), jnp.float32),
    )(aggp, degp, b)


def _tc_pool_body(x4_ref, w1_ref, c1_ref, w2_ref, c2_ref,
                  w3_ref, c3_ref, z_ref, acc):
    i = pl.program_id(0)

    @pl.when(i == 0)
    def _():
        acc[...] = jnp.zeros_like(acc)

    acc[...] += jnp.sum(x4_ref[...], axis=0, keepdims=True)

    @pl.when(i == N // _MB - 1)
    def _():
        pooled = acc[...] * (1.0 / N)
        z1 = _leaky(jnp.dot(pooled, w1_ref[...],
                            preferred_element_type=jnp.float32) + c1_ref[...])
        z2 = _leaky(jnp.dot(z1, w2_ref[...],
                            preferred_element_type=jnp.float32) + c2_ref[...])
        z_ref[...] = (jnp.dot(z2, w3_ref[...],
                              preferred_element_type=jnp.float32)
                      + c3_ref[...])


def _tc_pool(x4, w1, c1, w2, c2, w3, c3):
    full = lambda *shape: pl.BlockSpec(shape, lambda i: (0,) * len(shape))
    return pl.pallas_call(
        _tc_pool_body,
        grid=(N // _MB,),
        in_specs=[
            pl.BlockSpec((_MB, OUT), lambda i: (i, 0)),
            full(OUT, 16), full(1, 16),
            full(16, 8), full(1, 8),
            full(8, 1), full(1, 1),
        ],
        out_specs=full(1, 1),
        out_shape=jax.ShapeDtypeStruct((1, 1), jnp.float32),
        scratch_shapes=[pltpu.VMEM((1, OUT), jnp.float32)],
    )(x4, w1, c1, w2, c2, w3, c3)


FIN = 320


def kernel(x, edge_index, params):
    src = edge_index[0]
    dst = edge_index[1]

    src2 = src.reshape(NW, NCH, C)
    dst2 = dst.reshape(NW, NCH, C)
    src2p = src.reshape(NW, NCHP, CP)
    dst2p = dst.reshape(NW, NCHP, CP)
    zeros1 = jnp.zeros((ROWS_T,), jnp.float32)
    zerosm = jnp.zeros((ROWS_T, OUT), jnp.float32)

    spec_flat = x[:, :3].reshape(-1)
    pseudo_w, degp = _sc_prep(spec_flat, src2p, dst2p, zeros1)
    degp3 = degp.reshape(NC, NP, 1)

    gcs = [params["gc%d" % (i + 1)] for i in range(4)]
    w1 = jnp.stack([g["Kw1"].T for g in gcs])
    b1 = jnp.stack([g["Kb1"].reshape(16, 1) for g in gcs])
    w2 = jnp.stack([g["Kw2"].T for g in gcs])
    b2 = jnp.stack([g["Kb2"].reshape(8, 1) for g in gcs])
    al_all = _tc_alpha(pseudo_w, w1, b1, w2, b2)

    zblk = jnp.zeros((KER, OUT, OUT), jnp.float32)
    wflat = []
    for l in range(4):
        wk = gcs[l]["Wk"]
        parts = [wk[:, l * OUT:l * OUT + 128]]
        for m in range(1, 4):
            j = l - m
            parts.append(wk[:, j * OUT:(j + 1) * OUT] if 0 <= j <= l - 1
                         else zblk)
        wpad = jnp.concatenate(parts, axis=1)
        wflat.append(jnp.transpose(wpad, (1, 0, 2)).reshape(FIN, KER * OUT))
    wflat_all = jnp.stack(wflat)
    bias_all = jnp.stack([g["b"].reshape(1, OUT) for g in gcs])

    feats0 = jnp.concatenate(
        [x, jnp.zeros((N, FIN - 128), jnp.float32)], axis=1)

    lidx_all = jnp.broadcast_to(
        jnp.arange(4, dtype=jnp.int32)[:, None], (4, 16))

    def step(feats, xs_l):
        wflat_l, bias_l, lidx_l, l_idx = xs_l
        y = _tc_mm(feats, wflat_l)
        aggp, = _sc_edge(y, src2, dst2, al_all, lidx_l, zerosm)
        xl = _tc_act(aggp, degp3, bias_l)
        off = 128 + jnp.minimum(l_idx, 2) * OUT
        feats = lax.dynamic_update_slice(feats, xl, (0, off))
        return feats, xl

    _, xls = lax.scan(step, feats0,
                      (wflat_all, bias_all, lidx_all, jnp.arange(4)))
    x4 = xls[3]

    z = _tc_pool(x4,
                 params["lin1_w"], params["lin1_b"].reshape(1, 16),
                 params["lin2_w"], params["lin2_b"].reshape(1, 8),
                 params["lin3_w"], params["lin3_b"].reshape(1, 1))
    return z.reshape(1)
